# baseline (speedup 1.0000x reference)
import jax
import jax.numpy as jnp
from jax import lax
from jax.experimental import pallas as pl
from jax.experimental.pallas import tpu as pltpu
from jax.experimental.pallas import tpu_sc as plsc

NUM_CORES = 2
NUM_SUBCORES = 16
LANES = 16
NW = NUM_CORES * NUM_SUBCORES

BATCH = 16384
EMBED_K = 128
CHUNK = 128
B_PER_W = BATCH // NW
NCHUNK = B_PER_W // CHUNK
KREGS = EMBED_K // LANES

_BITREV4 = [0, 8, 4, 12, 2, 10, 6, 14, 1, 9, 5, 13, 3, 11, 7, 15]

_SHUF_DNUMS = lax.GatherDimensionNumbers(
    offset_dims=(), collapsed_slice_dims=(0,), start_index_map=(0,))


def _shuffle(x, idx):
    return lax.gather(x, idx[:, None], _SHUF_DNUMS, (1,),
                      mode=lax.GatherScatterMode.PROMISE_IN_BOUNDS)


def _mf_body(w_hbm, h_hbm, uidx_hbm, vidx_hbm, out_hbm,
             uidx_v, vidx_v, u_rows, v_rows, out_v,
             sem_u, sem_v, sem_i):
    wid = lax.axis_index("s") * NUM_CORES + lax.axis_index("c")
    base = wid * B_PER_W

    lane = lax.iota(jnp.int32, LANES)
    perms = [jnp.bitwise_xor(lane, s) for s in (8, 4, 2, 1)]
    masks = [(lane & (2 * s - 1)) < s for s in (8, 4, 2, 1)]
    bitrev = (((lane & 1) << 3) | ((lane & 2) << 1)
              | ((lane & 4) >> 1) | ((lane & 8) >> 3))

    ci_u = pltpu.async_copy(uidx_hbm.at[wid], uidx_v, sem_i)
    ci_v = pltpu.async_copy(vidx_hbm.at[wid], vidx_v, sem_i)
    ci_u.wait()
    ci_v.wait()

    def issue(ci):
        b = ci % 2
        cu = pltpu.async_copy(w_hbm.at[uidx_v.at[ci]], u_rows.at[b], sem_u[b])
        cv = pltpu.async_copy(h_hbm.at[vidx_v.at[ci]], v_rows.at[b], sem_v[b])
        return cu, cv

    copies = {0: issue(0)}
    for ci in range(NCHUNK):
        if ci + 1 < NCHUNK:
            copies[ci + 1] = issue(ci + 1)
        cu, cv = copies.pop(ci)
        cu.wait()
        cv.wait()
        b = ci % 2
        ub = u_rows.at[b]
        vb = v_rows.at[b]

        def dot16(row, ub=ub, vb=vb):
            prods = [ub[row, pl.ds(c * LANES, LANES)]
                     * vb[row, pl.ds(c * LANES, LANES)]
                     for c in range(KREGS)]
            while len(prods) > 1:
                prods = [prods[i] + prods[i + 1]
                         for i in range(0, len(prods) - 1, 2)]
            return prods[0]

        def merge(a, b2, r):
            return jnp.where(masks[r],
                             a + _shuffle(a, perms[r]),
                             b2 + _shuffle(b2, perms[r]))

        def reduce_range(lo, size):
            if size == 1:
                return dot16(lo)
            half = size // 2
            a = reduce_range(lo, half)
            b2 = reduce_range(lo + half, half)
            r = {2: 0, 4: 1, 8: 2, 16: 3}[size]
            return merge(a, b2, r)

        @plsc.parallel_loop(0, CHUNK // LANES, unroll=1)
        def group(g):
            packed = _shuffle(reduce_range(g * LANES, LANES), bitrev)
            out_v[pl.ds(g * LANES, LANES)] = packed

        pltpu.sync_copy(out_v, out_hbm.at[pl.ds(base + ci * CHUNK, CHUNK)])


@jax.jit
def _mf(w, h, uidx, vidx):
    return pl.kernel(
        _mf_body,
        out_type=jax.ShapeDtypeStruct((BATCH,), jnp.float32),
        mesh=plsc.VectorSubcoreMesh(core_axis_name="c", subcore_axis_name="s"),
        scratch_types=[
            pltpu.VMEM((NCHUNK, CHUNK), jnp.int32),
            pltpu.VMEM((NCHUNK, CHUNK), jnp.int32),
            pltpu.VMEM((2, CHUNK, EMBED_K), jnp.float32),
            pltpu.VMEM((2, CHUNK, EMBED_K), jnp.float32),
            pltpu.VMEM((CHUNK,), jnp.float32),
            [pltpu.SemaphoreType.DMA, pltpu.SemaphoreType.DMA],
            [pltpu.SemaphoreType.DMA, pltpu.SemaphoreType.DMA],
            pltpu.SemaphoreType.DMA,
        ],
    )(w, h, uidx, vidx)


def kernel(x, W, H):
    uidx = x[:, 0].astype(jnp.int32).reshape(NW, NCHUNK, CHUNK)
    vidx = x[:, 1].astype(jnp.int32).reshape(NW, NCHUNK, CHUNK)
    return _mf(W, H, uidx, vidx)


# speedup vs baseline: 1.1275x; 1.1275x over previous
import jax
import jax.numpy as jnp
from jax import lax
from jax.experimental import pallas as pl
from jax.experimental.pallas import tpu as pltpu
from jax.experimental.pallas import tpu_sc as plsc

NUM_CORES = 2
NUM_SUBCORES = 16
LANES = 16
NW = NUM_CORES * NUM_SUBCORES

BATCH = 16384
EMBED_K = 128
CHUNK = 128
B_PER_W = BATCH // NW
NCHUNK = B_PER_W // CHUNK
KREGS = EMBED_K // LANES

_BITREV4 = [0, 8, 4, 12, 2, 10, 6, 14, 1, 9, 5, 13, 3, 11, 7, 15]

_SHUF_DNUMS = lax.GatherDimensionNumbers(
    offset_dims=(), collapsed_slice_dims=(0,), start_index_map=(0,))


def _shuffle(x, idx):
    return lax.gather(x, idx[:, None], _SHUF_DNUMS, (1,),
                      mode=lax.GatherScatterMode.PROMISE_IN_BOUNDS)


def _mf_body(w_hbm, h_hbm, uidx_hbm, vidx_hbm, out_hbm,
             uidx_v, vidx_v, u_rows, v_rows, out_v,
             sem_u, sem_v, sem_i):
    wid = lax.axis_index("s") * NUM_CORES + lax.axis_index("c")
    base = wid * B_PER_W

    lane = lax.iota(jnp.int32, LANES)
    perms = [jnp.bitwise_xor(lane, s) for s in (8, 4, 2, 1)]
    masks = [(lane & (2 * s - 1)) < s for s in (8, 4, 2, 1)]
    bitrev = (((lane & 1) << 3) | ((lane & 2) << 1)
              | ((lane & 4) >> 1) | ((lane & 8) >> 3))

    ci_u = pltpu.async_copy(uidx_hbm.at[wid], uidx_v, sem_i)
    ci_v = pltpu.async_copy(vidx_hbm.at[wid], vidx_v, sem_i)
    ci_u.wait()
    ci_v.wait()

    def issue(ci):
        b = ci % 2
        cu = pltpu.async_copy(w_hbm.at[uidx_v.at[ci]], u_rows.at[b], sem_u[b])
        cv = pltpu.async_copy(h_hbm.at[vidx_v.at[ci]], v_rows.at[b], sem_v[b])
        return cu, cv

    copies = {0: issue(0)}
    for ci in range(NCHUNK):
        if ci + 1 < NCHUNK:
            copies[ci + 1] = issue(ci + 1)
        cu, cv = copies.pop(ci)
        cu.wait()
        cv.wait()
        b = ci % 2
        ub = u_rows.at[b]
        vb = v_rows.at[b]

        def dot16(row, ub=ub, vb=vb):
            prods = [ub[row, pl.ds(c * LANES, LANES)]
                     * vb[row, pl.ds(c * LANES, LANES)]
                     for c in range(KREGS)]
            while len(prods) > 1:
                prods = [prods[i] + prods[i + 1]
                         for i in range(0, len(prods) - 1, 2)]
            return prods[0]

        def merge(a, b2, r):
            return jnp.where(masks[r],
                             a + _shuffle(a, perms[r]),
                             b2 + _shuffle(b2, perms[r]))

        def reduce_range(lo, size):
            if size == 1:
                return dot16(lo)
            half = size // 2
            a = reduce_range(lo, half)
            b2 = reduce_range(lo + half, half)
            r = {2: 0, 4: 1, 8: 2, 16: 3}[size]
            return merge(a, b2, r)

        @plsc.parallel_loop(0, CHUNK // LANES, unroll=2)
        def group(g):
            packed = _shuffle(reduce_range(g * LANES, LANES), bitrev)
            out_v[pl.ds(g * LANES, LANES)] = packed

        pltpu.sync_copy(out_v, out_hbm.at[pl.ds(base + ci * CHUNK, CHUNK)])


@jax.jit
def _mf(w, h, uidx, vidx):
    return pl.kernel(
        _mf_body,
        out_type=jax.ShapeDtypeStruct((BATCH,), jnp.float32),
        mesh=plsc.VectorSubcoreMesh(core_axis_name="c", subcore_axis_name="s"),
        scratch_types=[
            pltpu.VMEM((NCHUNK, CHUNK), jnp.int32),
            pltpu.VMEM((NCHUNK, CHUNK), jnp.int32),
            pltpu.VMEM((2, CHUNK, EMBED_K), jnp.float32),
            pltpu.VMEM((2, CHUNK, EMBED_K), jnp.float32),
            pltpu.VMEM((CHUNK,), jnp.float32),
            [pltpu.SemaphoreType.DMA, pltpu.SemaphoreType.DMA],
            [pltpu.SemaphoreType.DMA, pltpu.SemaphoreType.DMA],
            pltpu.SemaphoreType.DMA,
        ],
    )(w, h, uidx, vidx)


def kernel(x, W, H):
    uidx = x[:, 0].astype(jnp.int32).reshape(NW, NCHUNK, CHUNK)
    vidx = x[:, 1].astype(jnp.int32).reshape(NW, NCHUNK, CHUNK)
    return _mf(W, H, uidx, vidx)


# speedup vs baseline: 1.2167x; 1.0791x over previous
import jax
import jax.numpy as jnp
from jax import lax
from jax.experimental import pallas as pl
from jax.experimental.pallas import tpu as pltpu
from jax.experimental.pallas import tpu_sc as plsc

NUM_CORES = 2
NUM_SUBCORES = 16
LANES = 16
NW = NUM_CORES * NUM_SUBCORES

BATCH = 16384
EMBED_K = 128
CHUNK = 128
B_PER_W = BATCH // NW
NCHUNK = B_PER_W // CHUNK
KREGS = EMBED_K // LANES

_BITREV4 = [0, 8, 4, 12, 2, 10, 6, 14, 1, 9, 5, 13, 3, 11, 7, 15]

_SHUF_DNUMS = lax.GatherDimensionNumbers(
    offset_dims=(), collapsed_slice_dims=(0,), start_index_map=(0,))


def _shuffle(x, idx):
    return lax.gather(x, idx[:, None], _SHUF_DNUMS, (1,),
                      mode=lax.GatherScatterMode.PROMISE_IN_BOUNDS)


def _mf_body(w_hbm, h_hbm, uidx_hbm, vidx_hbm, out_hbm,
             uidx_v, vidx_v, u_rows, v_rows, out_v,
             sem_u, sem_v, sem_i):
    wid = lax.axis_index("s") * NUM_CORES + lax.axis_index("c")
    base = wid * B_PER_W

    lane = lax.iota(jnp.int32, LANES)
    perms = [jnp.bitwise_xor(lane, s) for s in (8, 4, 2, 1)]
    masks = [(lane & (2 * s - 1)) < s for s in (8, 4, 2, 1)]
    bitrev = (((lane & 1) << 3) | ((lane & 2) << 1)
              | ((lane & 4) >> 1) | ((lane & 8) >> 3))

    ci_u = pltpu.async_copy(uidx_hbm.at[wid], uidx_v, sem_i)
    ci_v = pltpu.async_copy(vidx_hbm.at[wid], vidx_v, sem_i)
    ci_u.wait()
    ci_v.wait()

    def issue(ci):
        b = ci % 2
        cu = pltpu.async_copy(w_hbm.at[uidx_v.at[ci]], u_rows.at[b], sem_u[b])
        cv = pltpu.async_copy(h_hbm.at[vidx_v.at[ci]], v_rows.at[b], sem_v[b])
        return cu, cv

    copies = {0: issue(0)}
    for ci in range(NCHUNK):
        if ci + 1 < NCHUNK:
            copies[ci + 1] = issue(ci + 1)
        cu, cv = copies.pop(ci)
        cu.wait()
        cv.wait()
        b = ci % 2
        ub = u_rows.at[b]
        vb = v_rows.at[b]

        def dot16(row, ub=ub, vb=vb):
            prods = [ub[row, pl.ds(c * LANES, LANES)]
                     * vb[row, pl.ds(c * LANES, LANES)]
                     for c in range(KREGS)]
            while len(prods) > 1:
                prods = [prods[i] + prods[i + 1]
                         for i in range(0, len(prods) - 1, 2)]
            return prods[0]

        def merge(a, b2, r):
            return jnp.where(masks[r],
                             a + _shuffle(a, perms[r]),
                             b2 + _shuffle(b2, perms[r]))

        def reduce_range(lo, size):
            if size == 1:
                return dot16(lo)
            half = size // 2
            a = reduce_range(lo, half)
            b2 = reduce_range(lo + half, half)
            r = {2: 0, 4: 1, 8: 2, 16: 3}[size]
            return merge(a, b2, r)

        def group(g, _):
            packed = _shuffle(reduce_range(g * LANES, LANES), bitrev)
            out_v[pl.ds(g * LANES, LANES)] = packed
            return 0

        lax.fori_loop(0, CHUNK // LANES, group, 0)

        pltpu.sync_copy(out_v, out_hbm.at[pl.ds(base + ci * CHUNK, CHUNK)])


@jax.jit
def _mf(w, h, uidx, vidx):
    return pl.kernel(
        _mf_body,
        out_type=jax.ShapeDtypeStruct((BATCH,), jnp.float32),
        mesh=plsc.VectorSubcoreMesh(core_axis_name="c", subcore_axis_name="s"),
        scratch_types=[
            pltpu.VMEM((NCHUNK, CHUNK), jnp.int32),
            pltpu.VMEM((NCHUNK, CHUNK), jnp.int32),
            pltpu.VMEM((2, CHUNK, EMBED_K), jnp.float32),
            pltpu.VMEM((2, CHUNK, EMBED_K), jnp.float32),
            pltpu.VMEM((CHUNK,), jnp.float32),
            [pltpu.SemaphoreType.DMA, pltpu.SemaphoreType.DMA],
            [pltpu.SemaphoreType.DMA, pltpu.SemaphoreType.DMA],
            pltpu.SemaphoreType.DMA,
        ],
    )(w, h, uidx, vidx)


def kernel(x, W, H):
    uidx = x[:, 0].astype(jnp.int32).reshape(NW, NCHUNK, CHUNK)
    vidx = x[:, 1].astype(jnp.int32).reshape(NW, NCHUNK, CHUNK)
    return _mf(W, H, uidx, vidx)


# speedup vs baseline: 1.5016x; 1.2342x over previous
import jax
import jax.numpy as jnp
from jax import lax
from jax.experimental import pallas as pl
from jax.experimental.pallas import tpu as pltpu
from jax.experimental.pallas import tpu_sc as plsc

NUM_CORES = 2
NUM_SUBCORES = 16
LANES = 16
NW = NUM_CORES * NUM_SUBCORES

BATCH = 16384
EMBED_K = 128
CHUNK = 128
B_PER_W = BATCH // NW
NCHUNK = B_PER_W // CHUNK
KREGS = EMBED_K // LANES

_BITREV4 = [0, 8, 4, 12, 2, 10, 6, 14, 1, 9, 5, 13, 3, 11, 7, 15]

_SHUF_DNUMS = lax.GatherDimensionNumbers(
    offset_dims=(), collapsed_slice_dims=(0,), start_index_map=(0,))


def _shuffle(x, idx):
    return lax.gather(x, idx[:, None], _SHUF_DNUMS, (1,),
                      mode=lax.GatherScatterMode.PROMISE_IN_BOUNDS)


def _mf_body(w_hbm, h_hbm, uidx_hbm, vidx_hbm, out_hbm,
             uidx_v, vidx_v, u_rows, v_rows, out_v,
             sem_u, sem_v, sem_i):
    wid = lax.axis_index("s") * NUM_CORES + lax.axis_index("c")
    base = wid * B_PER_W

    lane = lax.iota(jnp.int32, LANES)
    perms = [jnp.bitwise_xor(lane, s) for s in (8, 4, 2, 1)]
    masks = [(lane & (2 * s - 1)) < s for s in (8, 4, 2, 1)]
    bitrev = (((lane & 1) << 3) | ((lane & 2) << 1)
              | ((lane & 4) >> 1) | ((lane & 8) >> 3))

    ci_u = pltpu.async_copy(uidx_hbm.at[wid], uidx_v, sem_i)
    ci_v = pltpu.async_copy(vidx_hbm.at[wid], vidx_v, sem_i)
    ci_u.wait()
    ci_v.wait()

    def issue(ci):
        b = ci % 2
        cu = pltpu.async_copy(w_hbm.at[uidx_v.at[ci]], u_rows.at[b], sem_u[b])
        cv = pltpu.async_copy(h_hbm.at[vidx_v.at[ci]], v_rows.at[b], sem_v[b])
        return cu, cv

    copies = {0: issue(0)}
    for ci in range(NCHUNK):
        if ci + 1 < NCHUNK:
            copies[ci + 1] = issue(ci + 1)
        cu, cv = copies.pop(ci)
        cu.wait()
        cv.wait()
        b = ci % 2
        ub = u_rows.at[b]
        vb = v_rows.at[b]

        def dot16(row, ub=ub, vb=vb):
            prods = [ub[row, pl.ds(c * LANES, LANES)]
                     * vb[row, pl.ds(c * LANES, LANES)]
                     for c in range(KREGS)]
            while len(prods) > 1:
                prods = [prods[i] + prods[i + 1]
                         for i in range(0, len(prods) - 1, 2)]
            return prods[0]

        def merge(a, b2, r):
            return jnp.where(masks[r],
                             a + _shuffle(a, perms[r]),
                             b2 + _shuffle(b2, perms[r]))

        def reduce_range(lo, size):
            if size == 1:
                return dot16(lo)
            half = size // 2
            a = reduce_range(lo, half)
            b2 = reduce_range(lo + half, half)
            r = {2: 0, 4: 1, 8: 2, 16: 3}[size]
            return merge(a, b2, r)

        def half_group(i, carry):
            a = reduce_range(i * (LANES // 2), LANES // 2)

            @pl.when((i & 1) == 1)
            def _():
                packed = _shuffle(merge(carry, a, 3), bitrev)
                out_v[pl.ds((i >> 1) * LANES, LANES)] = packed

            return a

        lax.fori_loop(0, 2 * (CHUNK // LANES), half_group,
                      jnp.zeros((LANES,), jnp.float32))

        pltpu.sync_copy(out_v, out_hbm.at[pl.ds(base + ci * CHUNK, CHUNK)])


@jax.jit
def _mf(w, h, uidx, vidx):
    return pl.kernel(
        _mf_body,
        out_type=jax.ShapeDtypeStruct((BATCH,), jnp.float32),
        mesh=plsc.VectorSubcoreMesh(core_axis_name="c", subcore_axis_name="s"),
        scratch_types=[
            pltpu.VMEM((NCHUNK, CHUNK), jnp.int32),
            pltpu.VMEM((NCHUNK, CHUNK), jnp.int32),
            pltpu.VMEM((2, CHUNK, EMBED_K), jnp.float32),
            pltpu.VMEM((2, CHUNK, EMBED_K), jnp.float32),
            pltpu.VMEM((CHUNK,), jnp.float32),
            [pltpu.SemaphoreType.DMA, pltpu.SemaphoreType.DMA],
            [pltpu.SemaphoreType.DMA, pltpu.SemaphoreType.DMA],
            pltpu.SemaphoreType.DMA,
        ],
    )(w, h, uidx, vidx)


def kernel(x, W, H):
    uidx = x[:, 0].astype(jnp.int32).reshape(NW, NCHUNK, CHUNK)
    vidx = x[:, 1].astype(jnp.int32).reshape(NW, NCHUNK, CHUNK)
    return _mf(W, H, uidx, vidx)


# speedup vs baseline: 1.5067x; 1.0034x over previous
import jax
import jax.numpy as jnp
from jax import lax
from jax.experimental import pallas as pl
from jax.experimental.pallas import tpu as pltpu
from jax.experimental.pallas import tpu_sc as plsc

NUM_CORES = 2
NUM_SUBCORES = 16
LANES = 16
NW = NUM_CORES * NUM_SUBCORES

BATCH = 16384
EMBED_K = 128
CHUNK = 128
B_PER_W = BATCH // NW
NCHUNK = B_PER_W // CHUNK
KREGS = EMBED_K // LANES

_BITREV4 = [0, 8, 4, 12, 2, 10, 6, 14, 1, 9, 5, 13, 3, 11, 7, 15]

_SHUF_DNUMS = lax.GatherDimensionNumbers(
    offset_dims=(), collapsed_slice_dims=(0,), start_index_map=(0,))


def _shuffle(x, idx):
    return lax.gather(x, idx[:, None], _SHUF_DNUMS, (1,),
                      mode=lax.GatherScatterMode.PROMISE_IN_BOUNDS)


def _mf_body(w_hbm, h_hbm, uidx_hbm, vidx_hbm, out_hbm,
             uidx_v, vidx_v, u_rows, v_rows, out_v,
             sem_u, sem_v, sem_i):
    wid = lax.axis_index("s") * NUM_CORES + lax.axis_index("c")
    base = wid * B_PER_W

    lane = lax.iota(jnp.int32, LANES)
    perms = [jnp.bitwise_xor(lane, s) for s in (8, 4, 2, 1)]
    masks = [(lane & (2 * s - 1)) < s for s in (8, 4, 2, 1)]
    bitrev = (((lane & 1) << 3) | ((lane & 2) << 1)
              | ((lane & 4) >> 1) | ((lane & 8) >> 3))

    ci_u = pltpu.async_copy(uidx_hbm.at[wid], uidx_v, sem_i)
    ci_v = pltpu.async_copy(vidx_hbm.at[wid], vidx_v, sem_i)
    ci_u.wait()
    ci_v.wait()

    def issue(ci):
        b = ci % 2
        cu = pltpu.async_copy(w_hbm.at[uidx_v.at[ci]], u_rows.at[b], sem_u[b])
        cv = pltpu.async_copy(h_hbm.at[vidx_v.at[ci]], v_rows.at[b], sem_v[b])
        return cu, cv

    copies = {0: issue(0)}
    for ci in range(NCHUNK):
        if ci + 1 < NCHUNK:
            copies[ci + 1] = issue(ci + 1)
        cu, cv = copies.pop(ci)
        cu.wait()
        cv.wait()
        b = ci % 2
        ub = u_rows.at[b]
        vb = v_rows.at[b]

        def dot16(row, ub=ub, vb=vb):
            prods = [ub[row, pl.ds(c * LANES, LANES)]
                     * vb[row, pl.ds(c * LANES, LANES)]
                     for c in range(KREGS)]
            while len(prods) > 1:
                prods = [prods[i] + prods[i + 1]
                         for i in range(0, len(prods) - 1, 2)]
            return prods[0]

        def merge(a, b2, r):
            return jnp.where(masks[r],
                             a + _shuffle(a, perms[r]),
                             b2 + _shuffle(b2, perms[r]))

        def reduce_range(lo, size):
            if size == 1:
                return dot16(lo)
            half = size // 2
            a = reduce_range(lo, half)
            b2 = reduce_range(lo + half, half)
            r = {2: 0, 4: 1, 8: 2, 16: 3}[size]
            return merge(a, b2, r)

        @plsc.parallel_loop(0, 2 * (CHUNK // LANES), unroll=2,
                            carry=jnp.zeros((LANES,), jnp.float32))
        def half_group(i, carry):
            a = reduce_range(i * (LANES // 2), LANES // 2)

            @pl.when((i & 1) == 1)
            def _():
                packed = _shuffle(merge(carry, a, 3), bitrev)
                out_v[pl.ds((i >> 1) * LANES, LANES)] = packed

            return a

        pltpu.sync_copy(out_v, out_hbm.at[pl.ds(base + ci * CHUNK, CHUNK)])


@jax.jit
def _mf(w, h, uidx, vidx):
    return pl.kernel(
        _mf_body,
        out_type=jax.ShapeDtypeStruct((BATCH,), jnp.float32),
        mesh=plsc.VectorSubcoreMesh(core_axis_name="c", subcore_axis_name="s"),
        scratch_types=[
            pltpu.VMEM((NCHUNK, CHUNK), jnp.int32),
            pltpu.VMEM((NCHUNK, CHUNK), jnp.int32),
            pltpu.VMEM((2, CHUNK, EMBED_K), jnp.float32),
            pltpu.VMEM((2, CHUNK, EMBED_K), jnp.float32),
            pltpu.VMEM((CHUNK,), jnp.float32),
            [pltpu.SemaphoreType.DMA, pltpu.SemaphoreType.DMA],
            [pltpu.SemaphoreType.DMA, pltpu.SemaphoreType.DMA],
            pltpu.SemaphoreType.DMA,
        ],
    )(w, h, uidx, vidx)


def kernel(x, W, H):
    uidx = x[:, 0].astype(jnp.int32).reshape(NW, NCHUNK, CHUNK)
    vidx = x[:, 1].astype(jnp.int32).reshape(NW, NCHUNK, CHUNK)
    return _mf(W, H, uidx, vidx)


# speedup vs baseline: 1.5661x; 1.0395x over previous
import jax
import jax.numpy as jnp
from jax import lax
from jax.experimental import pallas as pl
from jax.experimental.pallas import tpu as pltpu
from jax.experimental.pallas import tpu_sc as plsc

NUM_CORES = 2
NUM_SUBCORES = 16
LANES = 16
NW = NUM_CORES * NUM_SUBCORES

BATCH = 16384
EMBED_K = 128
CHUNK = 128
B_PER_W = BATCH // NW
NCHUNK = B_PER_W // CHUNK
KREGS = EMBED_K // LANES

_BITREV4 = [0, 8, 4, 12, 2, 10, 6, 14, 1, 9, 5, 13, 3, 11, 7, 15]

_SHUF_DNUMS = lax.GatherDimensionNumbers(
    offset_dims=(), collapsed_slice_dims=(0,), start_index_map=(0,))


def _shuffle(x, idx):
    return lax.gather(x, idx[:, None], _SHUF_DNUMS, (1,),
                      mode=lax.GatherScatterMode.PROMISE_IN_BOUNDS)


def _mf_body(w_hbm, h_hbm, uidx_hbm, vidx_hbm, out_hbm,
             uidx_v, vidx_v, u_rows, v_rows, out_v,
             sem_u, sem_v, sem_i, sem_o):
    wid = lax.axis_index("s") * NUM_CORES + lax.axis_index("c")
    base = wid * B_PER_W

    lane = lax.iota(jnp.int32, LANES)
    perms = [jnp.bitwise_xor(lane, s) for s in (8, 4, 2, 1)]
    masks = [(lane & (2 * s - 1)) < s for s in (8, 4, 2, 1)]
    bitrev = (((lane & 1) << 3) | ((lane & 2) << 1)
              | ((lane & 4) >> 1) | ((lane & 8) >> 3))

    c0_u = pltpu.async_copy(uidx_hbm.at[wid].at[0], uidx_v.at[0], sem_i)
    c0_v = pltpu.async_copy(vidx_hbm.at[wid].at[0], vidx_v.at[0], sem_i)
    c0_u.wait()
    c0_v.wait()

    def issue(ci):
        b = ci % 2
        cu = pltpu.async_copy(w_hbm.at[uidx_v.at[ci]], u_rows.at[b], sem_u[b])
        cv = pltpu.async_copy(h_hbm.at[vidx_v.at[ci]], v_rows.at[b], sem_v[b])
        return cu, cv

    copies = {0: issue(0)}
    cr_u = pltpu.async_copy(uidx_hbm.at[wid].at[pl.ds(1, NCHUNK - 1)],
                            uidx_v.at[pl.ds(1, NCHUNK - 1)], sem_i)
    cr_v = pltpu.async_copy(vidx_hbm.at[wid].at[pl.ds(1, NCHUNK - 1)],
                            vidx_v.at[pl.ds(1, NCHUNK - 1)], sem_i)
    cr_u.wait()
    cr_v.wait()

    out_copies = {}
    for ci in range(NCHUNK):
        if ci + 1 < NCHUNK:
            copies[ci + 1] = issue(ci + 1)
        cu, cv = copies.pop(ci)
        cu.wait()
        cv.wait()
        b = ci % 2
        ub = u_rows.at[b]
        vb = v_rows.at[b]
        if ci >= 2:
            out_copies.pop(ci - 2).wait()

        def dot16(row, ub=ub, vb=vb):
            prods = [ub[row, pl.ds(c * LANES, LANES)]
                     * vb[row, pl.ds(c * LANES, LANES)]
                     for c in range(KREGS)]
            while len(prods) > 1:
                prods = [prods[i] + prods[i + 1]
                         for i in range(0, len(prods) - 1, 2)]
            return prods[0]

        def merge(a, b2, r):
            return jnp.where(masks[r],
                             a + _shuffle(a, perms[r]),
                             b2 + _shuffle(b2, perms[r]))

        def reduce_range(lo, size):
            if size == 1:
                return dot16(lo)
            half = size // 2
            a = reduce_range(lo, half)
            b2 = reduce_range(lo + half, half)
            r = {2: 0, 4: 1, 8: 2, 16: 3}[size]
            return merge(a, b2, r)

        @plsc.parallel_loop(0, 2 * (CHUNK // LANES), unroll=2,
                            carry=jnp.zeros((LANES,), jnp.float32))
        def half_group(i, carry):
            a = reduce_range(i * (LANES // 2), LANES // 2)

            @pl.when((i & 1) == 1)
            def _():
                packed = _shuffle(merge(carry, a, 3), bitrev)
                out_v[b, pl.ds((i >> 1) * LANES, LANES)] = packed

            return a

        out_copies[ci] = pltpu.async_copy(
            out_v.at[b], out_hbm.at[pl.ds(base + ci * CHUNK, CHUNK)], sem_o[b])
    for ci in sorted(out_copies):
        out_copies.pop(ci).wait()


@jax.jit
def _mf(w, h, uidx, vidx):
    return pl.kernel(
        _mf_body,
        out_type=jax.ShapeDtypeStruct((BATCH,), jnp.float32),
        mesh=plsc.VectorSubcoreMesh(core_axis_name="c", subcore_axis_name="s"),
        scratch_types=[
            pltpu.VMEM((NCHUNK, CHUNK), jnp.int32),
            pltpu.VMEM((NCHUNK, CHUNK), jnp.int32),
            pltpu.VMEM((2, CHUNK, EMBED_K), jnp.float32),
            pltpu.VMEM((2, CHUNK, EMBED_K), jnp.float32),
            pltpu.VMEM((2, CHUNK), jnp.float32),
            [pltpu.SemaphoreType.DMA, pltpu.SemaphoreType.DMA],
            [pltpu.SemaphoreType.DMA, pltpu.SemaphoreType.DMA],
            pltpu.SemaphoreType.DMA,
            [pltpu.SemaphoreType.DMA, pltpu.SemaphoreType.DMA],
        ],
    )(w, h, uidx, vidx)


def kernel(x, W, H):
    uidx = x[:, 0].astype(jnp.int32).reshape(NW, NCHUNK, CHUNK)
    vidx = x[:, 1].astype(jnp.int32).reshape(NW, NCHUNK, CHUNK)
    return _mf(W, H, uidx, vidx)


# speedup vs baseline: 1.6341x; 1.0434x over previous
import jax
import jax.numpy as jnp
from jax import lax
from jax.experimental import pallas as pl
from jax.experimental.pallas import tpu as pltpu
from jax.experimental.pallas import tpu_sc as plsc

NUM_CORES = 2
NUM_SUBCORES = 16
LANES = 16
NW = NUM_CORES * NUM_SUBCORES

BATCH = 16384
EMBED_K = 128
CHUNK = 128
B_PER_W = BATCH // NW
NCHUNK = B_PER_W // CHUNK
KREGS = EMBED_K // LANES

_BITREV4 = [0, 8, 4, 12, 2, 10, 6, 14, 1, 9, 5, 13, 3, 11, 7, 15]

_SHUF_DNUMS = lax.GatherDimensionNumbers(
    offset_dims=(), collapsed_slice_dims=(0,), start_index_map=(0,))


def _shuffle(x, idx):
    return lax.gather(x, idx[:, None], _SHUF_DNUMS, (1,),
                      mode=lax.GatherScatterMode.PROMISE_IN_BOUNDS)


def _mf_body(w_hbm, h_hbm, uidx_hbm, vidx_hbm, out_hbm,
             uidx_v, vidx_v, u_rows, v_rows, out_v,
             sem_u, sem_v, sem_i, sem_o):
    wid = lax.axis_index("s") * NUM_CORES + lax.axis_index("c")
    base = wid * B_PER_W

    lane = lax.iota(jnp.int32, LANES)
    perms = [jnp.bitwise_xor(lane, s) for s in (8, 4, 2, 1)]
    masks = [(lane & (2 * s - 1)) < s for s in (8, 4, 2, 1)]
    bitrev = (((lane & 1) << 3) | ((lane & 2) << 1)
              | ((lane & 4) >> 1) | ((lane & 8) >> 3))

    c0_u = pltpu.async_copy(uidx_hbm.at[wid].at[0], uidx_v.at[0], sem_i)
    c0_v = pltpu.async_copy(vidx_hbm.at[wid].at[0], vidx_v.at[0], sem_i)
    c0_u.wait()
    c0_v.wait()

    def issue(ci):
        b = ci % 2
        cu = pltpu.async_copy(w_hbm.at[uidx_v.at[ci]], u_rows.at[b], sem_u[b])
        cv = pltpu.async_copy(h_hbm.at[vidx_v.at[ci]], v_rows.at[b], sem_v[b])
        return cu, cv

    copies = {0: issue(0)}
    cr_u = pltpu.async_copy(uidx_hbm.at[wid].at[pl.ds(1, NCHUNK - 1)],
                            uidx_v.at[pl.ds(1, NCHUNK - 1)], sem_i)
    cr_v = pltpu.async_copy(vidx_hbm.at[wid].at[pl.ds(1, NCHUNK - 1)],
                            vidx_v.at[pl.ds(1, NCHUNK - 1)], sem_i)
    cr_u.wait()
    cr_v.wait()

    out_copies = {}
    for ci in range(NCHUNK):
        if ci + 1 < NCHUNK:
            copies[ci + 1] = issue(ci + 1)
        cu, cv = copies.pop(ci)
        cu.wait()
        cv.wait()
        b = ci % 2
        ub = u_rows.at[b]
        vb = v_rows.at[b]
        if ci >= 2:
            out_copies.pop(ci - 2).wait()

        def dot16(row, ub=ub, vb=vb):
            acc0 = ub[row, pl.ds(0, LANES)] * vb[row, pl.ds(0, LANES)]
            acc1 = ub[row, pl.ds(LANES, LANES)] * vb[row, pl.ds(LANES, LANES)]
            for c in range(2, KREGS, 2):
                acc0 = acc0 + (ub[row, pl.ds(c * LANES, LANES)]
                               * vb[row, pl.ds(c * LANES, LANES)])
                acc1 = acc1 + (ub[row, pl.ds((c + 1) * LANES, LANES)]
                               * vb[row, pl.ds((c + 1) * LANES, LANES)])
            return acc0 + acc1

        def merge(a, b2, r):
            return jnp.where(masks[r],
                             a + _shuffle(a, perms[r]),
                             b2 + _shuffle(b2, perms[r]))

        def reduce_range(lo, size):
            if size == 1:
                return dot16(lo)
            half = size // 2
            a = reduce_range(lo, half)
            b2 = reduce_range(lo + half, half)
            r = {2: 0, 4: 1, 8: 2, 16: 3}[size]
            return merge(a, b2, r)

        @plsc.parallel_loop(0, 2 * (CHUNK // LANES), unroll=2,
                            carry=jnp.zeros((LANES,), jnp.float32))
        def half_group(i, carry):
            a = reduce_range(i * (LANES // 2), LANES // 2)

            @pl.when((i & 1) == 1)
            def _():
                packed = _shuffle(merge(carry, a, 3), bitrev)
                out_v[b, pl.ds((i >> 1) * LANES, LANES)] = packed

            return a

        out_copies[ci] = pltpu.async_copy(
            out_v.at[b], out_hbm.at[pl.ds(base + ci * CHUNK, CHUNK)], sem_o[b])
    for ci in sorted(out_copies):
        out_copies.pop(ci).wait()


@jax.jit
def _mf(w, h, uidx, vidx):
    return pl.kernel(
        _mf_body,
        out_type=jax.ShapeDtypeStruct((BATCH,), jnp.float32),
        mesh=plsc.VectorSubcoreMesh(core_axis_name="c", subcore_axis_name="s"),
        scratch_types=[
            pltpu.VMEM((NCHUNK, CHUNK), jnp.int32),
            pltpu.VMEM((NCHUNK, CHUNK), jnp.int32),
            pltpu.VMEM((2, CHUNK, EMBED_K), jnp.float32),
            pltpu.VMEM((2, CHUNK, EMBED_K), jnp.float32),
            pltpu.VMEM((2, CHUNK), jnp.float32),
            [pltpu.SemaphoreType.DMA, pltpu.SemaphoreType.DMA],
            [pltpu.SemaphoreType.DMA, pltpu.SemaphoreType.DMA],
            pltpu.SemaphoreType.DMA,
            [pltpu.SemaphoreType.DMA, pltpu.SemaphoreType.DMA],
        ],
    )(w, h, uidx, vidx)


def kernel(x, W, H):
    uidx = x[:, 0].astype(jnp.int32).reshape(NW, NCHUNK, CHUNK)
    vidx = x[:, 1].astype(jnp.int32).reshape(NW, NCHUNK, CHUNK)
    return _mf(W, H, uidx, vidx)
